# baseline (speedup 1.0000x reference)
import functools

import jax
import jax.numpy as jnp
import numpy as np
from jax import lax
from jax.experimental import pallas as pl
from jax.experimental.pallas import tpu as pltpu
from jax.experimental.pallas import tpu_sc as plsc

NC = 2
NS = 16
EK = 128
R = 4

_MESH = dict(core_axis_name="c", subcore_axis_name="s")


def _zero_acc(z_hbm, acc_sh, sid, RPT, TAIL):
    r0 = sid * RPT
    pltpu.sync_copy(z_hbm.at[pl.ds(r0, RPT)], acc_sh.at[pl.ds(r0, RPT)])
    if TAIL:
        @pl.when(sid == NS - 1)
        def _():
            pltpu.sync_copy(z_hbm.at[pl.ds(NS * RPT, TAIL)],
                            acc_sh.at[pl.ds(NS * RPT, TAIL)])


def _write_acc(acc_sh, out_hbm, cid, sid, RPT, TAIL):
    r0 = sid * RPT
    pltpu.sync_copy(acc_sh.at[pl.ds(r0, RPT)],
                    out_hbm.at[cid, pl.ds(r0, RPT)])
    if TAIL:
        @pl.when(sid == NS - 1)
        def _():
            pltpu.sync_copy(acc_sh.at[pl.ds(NS * RPT, TAIL)],
                            out_hbm.at[cid, pl.ds(NS * RPT, TAIL)])


def _sc_degree(edges, z128, ones128):
    _, E = edges.shape
    N, W = z128.shape
    NT = NC * NS
    NCH = E // EK
    FULL = NCH // NT
    EXTRA = NCH - FULL * NT
    RPT = (N // NS) // 8 * 8
    TAIL = N - NS * RPT
    CH = FULL // R

    mesh = plsc.VectorSubcoreMesh(**_MESH)

    @functools.partial(
        pl.kernel, mesh=mesh,
        out_type=jax.ShapeDtypeStruct((NC, N, W), jnp.float32),
        scratch_types=[
            pltpu.VMEM((FULL, EK), jnp.int32),
            pltpu.VMEM((EK,), jnp.int32),
            pltpu.VMEM((EK, W), jnp.float32),
            pltpu.VMEM_SHARED((N, W), jnp.float32),
            pltpu.SemaphoreType.DMA,
        ] + [pltpu.SemaphoreType.DMA] * R)
    def deg_kernel(edges_hbm, z_hbm, ones_hbm, out_hbm,
                   di2, dix_v, ones_v, acc_sh, m, *ss):
        dst_hbm = edges_hbm.at[1]
        cid = lax.axis_index("c")
        sid = lax.axis_index("s")
        w = cid * NS + sid
        descs = [pltpu.async_copy(dst_hbm.at[pl.ds((w + NT * j) * EK, EK)],
                                  di2.at[j], m) for j in range(FULL)]
        _zero_acc(z_hbm, acc_sh, sid, RPT, TAIL)
        pltpu.sync_copy(ones_hbm.at[pl.ds(0, EK)], ones_v)
        for d in descs:
            d.wait()
        plsc.subcore_barrier()

        for c in range(R):
            pltpu.async_copy(ones_v, acc_sh.at[di2.at[c]], ss[c], add=True)

        @pl.loop(1, CH)
        def _(j):
            for b in range(R):
                c = j * R + b
                pltpu.make_async_copy(ones_v, acc_sh.at[di2.at[c - R]],
                                      ss[b]).wait()
                pltpu.async_copy(ones_v, acc_sh.at[di2.at[c]], ss[b],
                                 add=True)

        for c in range(CH * R, FULL):
            pltpu.make_async_copy(ones_v, acc_sh.at[di2.at[c - R]],
                                  ss[c % R]).wait()
            pltpu.async_copy(ones_v, acc_sh.at[di2.at[c]], ss[c % R],
                             add=True)
        for c in range(FULL - R, FULL):
            pltpu.make_async_copy(ones_v, acc_sh.at[di2.at[c]],
                                  ss[c % R]).wait()
        if EXTRA:
            @pl.when(w < EXTRA)
            def _():
                pltpu.sync_copy(
                    dst_hbm.at[pl.ds((FULL * NT + w) * EK, EK)], dix_v)
                pltpu.sync_copy(ones_v, acc_sh.at[dix_v], add=True)
        plsc.subcore_barrier()
        _write_acc(acc_sh, out_hbm, cid, sid, RPT, TAIL)

    return deg_kernel(edges, z128, ones128)


def _sc_scatter(h, edges, z128):
    N, Dm = h.shape
    _, E = edges.shape
    NT = NC * NS
    NCH = E // EK
    FULL = NCH // NT
    EXTRA = NCH - FULL * NT
    RPT = (N // NS) // 8 * 8
    TAIL = N - NS * RPT
    KR = 2
    GROUPS = FULL // KR
    assert FULL % KR == 0 and GROUPS >= 3

    mesh = plsc.VectorSubcoreMesh(**_MESH)

    @functools.partial(
        pl.kernel, mesh=mesh,
        out_type=jax.ShapeDtypeStruct((NC, N, Dm), jnp.float32),
        scratch_types=(
            [pltpu.VMEM((EK,), jnp.int32) for _ in range(KR)] +
            [pltpu.VMEM((EK,), jnp.int32) for _ in range(KR)] +
            [pltpu.VMEM((EK, Dm), jnp.float32) for _ in range(KR)] +
            [pltpu.VMEM_SHARED((N, Dm), jnp.float32)] +
            [pltpu.SemaphoreType.DMA] * (4 * KR)))
    def scat_kernel(h_hbm, edges_hbm, z_hbm, out_hbm, *rest):
        src_hbm = edges_hbm.at[0]
        dst_hbm = edges_hbm.at[1]
        si = rest[0:KR]
        di = rest[KR:2 * KR]
        rows = rest[2 * KR:3 * KR]
        acc_sh = rest[3 * KR]
        iss = rest[3 * KR + 1:3 * KR + 1 + KR]
        dss = rest[3 * KR + 1 + KR:3 * KR + 1 + 2 * KR]
        gss = rest[3 * KR + 1 + 2 * KR:3 * KR + 1 + 3 * KR]
        sss = rest[3 * KR + 1 + 3 * KR:]
        cid = lax.axis_index("c")
        sid = lax.axis_index("s")
        w = cid * NS + sid

        def e0(c):
            return (w + NT * c) * EK

        def fire_si(c, b):
            pltpu.async_copy(src_hbm.at[pl.ds(e0(c), EK)], si[b], iss[b])

        def wait_si(c, b):
            pltpu.make_async_copy(src_hbm.at[pl.ds(e0(c), EK)], si[b],
                                  iss[b]).wait()

        def fire_di(c, b):
            pltpu.async_copy(dst_hbm.at[pl.ds(e0(c), EK)], di[b], dss[b])

        def wait_di(c, b):
            pltpu.make_async_copy(dst_hbm.at[pl.ds(e0(c), EK)], di[b],
                                  dss[b]).wait()

        def fire_g(b):
            pltpu.async_copy(h_hbm.at[si[b]], rows[b], gss[b])

        def wait_g(b):
            pltpu.make_async_copy(h_hbm.at[si[b]], rows[b], gss[b]).wait()

        def fire_s(b):
            pltpu.async_copy(rows[b], acc_sh.at[di[b]], sss[b], add=True)

        def wait_s(b):
            pltpu.make_async_copy(rows[b], acc_sh.at[di[b]], sss[b]).wait()

        fire_si(0, 0)
        fire_si(1, 1)
        fire_di(0, 0)
        _zero_acc(z_hbm, acc_sh, sid, RPT, TAIL)
        wait_si(0, 0)
        fire_g(0)
        plsc.subcore_barrier()

        def slot(c, b, first=False, fire_next=True, fire_next2=True):
            b1 = 1 - b
            if not first:
                wait_s(b1)
            if fire_next:
                fire_di(c + 1, b1)
                wait_si(c + 1, b1)
                fire_g(b1)
            wait_g(b)
            if fire_next2:
                fire_si(c + 2, b)
            wait_di(c, b)
            fire_s(b)

        slot(0, 0, first=True)
        slot(1, 1)

        @pl.loop(1, GROUPS - 1)
        def _(j):
            slot(KR * j, 0)
            slot(KR * j + 1, 1)

        slot(FULL - 2, 0, fire_next2=False)
        slot(FULL - 1, 1, fire_next=False, fire_next2=False)
        wait_s(1)
        if EXTRA:
            @pl.when(w < EXTRA)
            def _():
                ex0 = (FULL * NT + w) * EK
                pltpu.sync_copy(src_hbm.at[pl.ds(ex0, EK)], si[0])
                pltpu.sync_copy(dst_hbm.at[pl.ds(ex0, EK)], di[0])
                pltpu.sync_copy(h_hbm.at[si[0]], rows[0])
                pltpu.sync_copy(rows[0], acc_sh.at[di[0]], add=True)
        plsc.subcore_barrier()
        _write_acc(acc_sh, out_hbm, cid, sid, RPT, TAIL)

    return scat_kernel(h, edges, z128)


def _dot(a, b):
    return lax.dot_general(a, b, (((1,), (0,)), ((), ())),
                           precision=lax.Precision.DEFAULT,
                           preferred_element_type=jnp.float32)


BM = 1000


def _row_spec(Dm):
    return pl.BlockSpec((BM, Dm), lambda i: (i, 0))


def _pair_spec(Dm):
    return pl.BlockSpec((NC, BM, Dm), lambda i: (0, i, 0))


def _full_spec(a, b):
    return pl.BlockSpec((a, b), lambda i: (0, 0))


def _tc_matmul(x, W):
    N = x.shape[0]

    def body(x_ref, w_ref, o_ref):
        o_ref[...] = _dot(x_ref[...], w_ref[...])

    return pl.pallas_call(
        body,
        grid=(N // BM,),
        in_specs=[_row_spec(x.shape[1]), _full_spec(*W.shape)],
        out_specs=_row_spec(W.shape[1]),
        out_shape=jax.ShapeDtypeStruct((N, W.shape[1]), jnp.float32),
    )(x, W)


def _tc_prep(P1, deg):
    N, Dm = P1.shape

    def body(p_ref, deg_ref, hp_ref, dis_ref):
        d = deg_ref[0][:, 0:1] + deg_ref[1][:, 0:1] + 1.0
        dis = jnp.broadcast_to(lax.rsqrt(d), (BM, Dm))
        dis_ref[...] = dis
        hp_ref[...] = p_ref[...] * dis

    return pl.pallas_call(
        body,
        grid=(N // BM,),
        in_specs=[_row_spec(Dm), _pair_spec(Dm)],
        out_specs=[_row_spec(Dm), _row_spec(Dm)],
        out_shape=[jax.ShapeDtypeStruct((N, Dm), jnp.float32),
                   jax.ShapeDtypeStruct((N, Dm), jnp.float32)],
    )(P1, deg)


def _tc_mid(acc, hp, dis, b1, W2):
    N, Dm = hp.shape

    def body(acc_ref, hp_ref, dis_ref, b_ref, w_ref, o_ref):
        g = jnp.maximum(
            dis_ref[...] * (acc_ref[0] + acc_ref[1] + hp_ref[...])
            + b_ref[...], 0.0)
        o_ref[...] = _dot(g, w_ref[...]) * dis_ref[...]

    return pl.pallas_call(
        body,
        grid=(N // BM,),
        in_specs=[_pair_spec(Dm), _row_spec(Dm), _row_spec(Dm),
                  _full_spec(1, Dm), _full_spec(*W2.shape)],
        out_specs=_row_spec(W2.shape[1]),
        out_shape=jax.ShapeDtypeStruct((N, W2.shape[1]), jnp.float32),
    )(acc, hp, dis, b1, W2)


def _tc_final(acc, hp, dis, b2, Wl, bl):
    N, Dm = hp.shape

    def body(acc_ref, hp_ref, dis_ref, b_ref, w_ref, bl_ref, o_ref):
        g = jnp.maximum(
            dis_ref[...] * (acc_ref[0] + acc_ref[1] + hp_ref[...])
            + b_ref[...], 0.0)
        o_ref[...] = _dot(g, w_ref[...]) + bl_ref[...]

    return pl.pallas_call(
        body,
        grid=(N // BM,),
        in_specs=[_pair_spec(Dm), _row_spec(Dm), _row_spec(Dm),
                  _full_spec(1, Dm), _full_spec(*Wl.shape),
                  _full_spec(1, Wl.shape[1])],
        out_specs=_row_spec(Wl.shape[1]),
        out_shape=jax.ShapeDtypeStruct((N, Wl.shape[1]), jnp.float32),
    )(acc, hp, dis, b2, Wl, bl)


def kernel(x, edge_index, W1, b1, W2, b2, Wl, bl):
    N, D = x.shape
    edges = edge_index if edge_index.dtype == jnp.int32 \
        else edge_index.astype(jnp.int32)
    ones128 = jnp.asarray(np.ones((EK, 128), np.float32))
    z128 = jnp.asarray(np.zeros((N, W1.shape[1]), np.float32))

    deg = _sc_degree(edges, z128, ones128)
    P1 = _tc_matmul(x, W1)
    h1p, disb = _tc_prep(P1, deg)
    acc1 = _sc_scatter(h1p, edges, z128)
    h2p = _tc_mid(acc1, h1p, disb, b1.reshape(1, -1), W2)
    acc2 = _sc_scatter(h2p, edges, z128)
    out = _tc_final(acc2, h2p, disb, b2.reshape(1, -1),
                    Wl, bl.reshape(1, -1))
    return out


# speedup vs baseline: 1.0280x; 1.0280x over previous
import functools

import jax
import jax.numpy as jnp
import numpy as np
from jax import lax
from jax.experimental import pallas as pl
from jax.experimental.pallas import tpu as pltpu
from jax.experimental.pallas import tpu_sc as plsc

NC = 2
NS = 16
EK = 128
R = 4

_MESH = dict(core_axis_name="c", subcore_axis_name="s")


def _zero_acc(z_hbm, acc_sh, sid, RPT, TAIL):
    r0 = sid * RPT
    pltpu.sync_copy(z_hbm.at[pl.ds(r0, RPT)], acc_sh.at[pl.ds(r0, RPT)])
    if TAIL:
        @pl.when(sid == NS - 1)
        def _():
            pltpu.sync_copy(z_hbm.at[pl.ds(NS * RPT, TAIL)],
                            acc_sh.at[pl.ds(NS * RPT, TAIL)])


def _write_acc(acc_sh, out_hbm, cid, sid, RPT, TAIL):
    r0 = sid * RPT
    pltpu.sync_copy(acc_sh.at[pl.ds(r0, RPT)],
                    out_hbm.at[cid, pl.ds(r0, RPT)])
    if TAIL:
        @pl.when(sid == NS - 1)
        def _():
            pltpu.sync_copy(acc_sh.at[pl.ds(NS * RPT, TAIL)],
                            out_hbm.at[cid, pl.ds(NS * RPT, TAIL)])


def _sc_degree(edges, z128, ones128):
    _, E = edges.shape
    N, W = z128.shape
    NT = NC * NS
    NCH = E // EK
    FULL = NCH // NT
    EXTRA = NCH - FULL * NT
    RPT = (N // NS) // 8 * 8
    TAIL = N - NS * RPT
    CH = FULL // R

    mesh = plsc.VectorSubcoreMesh(**_MESH)

    @functools.partial(
        pl.kernel, mesh=mesh,
        out_type=jax.ShapeDtypeStruct((NC, N, W), jnp.float32),
        scratch_types=[
            pltpu.VMEM((FULL, EK), jnp.int32),
            pltpu.VMEM((EK,), jnp.int32),
            pltpu.VMEM((EK, W), jnp.float32),
            pltpu.VMEM_SHARED((N, W), jnp.float32),
            pltpu.SemaphoreType.DMA,
        ] + [pltpu.SemaphoreType.DMA] * R)
    def deg_kernel(edges_hbm, z_hbm, ones_hbm, out_hbm,
                   di2, dix_v, ones_v, acc_sh, m, *ss):
        dst_hbm = edges_hbm.at[1]
        cid = lax.axis_index("c")
        sid = lax.axis_index("s")
        w = cid * NS + sid
        descs = [pltpu.async_copy(dst_hbm.at[pl.ds((w + NT * j) * EK, EK)],
                                  di2.at[j], m) for j in range(FULL)]
        _zero_acc(z_hbm, acc_sh, sid, RPT, TAIL)
        pltpu.sync_copy(ones_hbm.at[pl.ds(0, EK)], ones_v)
        for d in descs:
            d.wait()
        plsc.subcore_barrier()

        for c in range(R):
            pltpu.async_copy(ones_v, acc_sh.at[di2.at[c]], ss[c], add=True)

        @pl.loop(1, CH)
        def _(j):
            for b in range(R):
                c = j * R + b
                pltpu.make_async_copy(ones_v, acc_sh.at[di2.at[c - R]],
                                      ss[b]).wait()
                pltpu.async_copy(ones_v, acc_sh.at[di2.at[c]], ss[b],
                                 add=True)

        for c in range(CH * R, FULL):
            pltpu.make_async_copy(ones_v, acc_sh.at[di2.at[c - R]],
                                  ss[c % R]).wait()
            pltpu.async_copy(ones_v, acc_sh.at[di2.at[c]], ss[c % R],
                             add=True)
        for c in range(FULL - R, FULL):
            pltpu.make_async_copy(ones_v, acc_sh.at[di2.at[c]],
                                  ss[c % R]).wait()
        if EXTRA:
            @pl.when(w < EXTRA)
            def _():
                pltpu.sync_copy(
                    dst_hbm.at[pl.ds((FULL * NT + w) * EK, EK)], dix_v)
                pltpu.sync_copy(ones_v, acc_sh.at[dix_v], add=True)
        plsc.subcore_barrier()
        _write_acc(acc_sh, out_hbm, cid, sid, RPT, TAIL)

    return deg_kernel(edges, z128, ones128)


def _sc_scatter(h, edges, z128):
    N, Dm = h.shape
    _, E = edges.shape
    NT = NC * NS
    NCH = E // EK
    FULL = NCH // NT
    EXTRA = NCH - FULL * NT
    RPT = (N // NS) // 8 * 8
    TAIL = N - NS * RPT
    KR = 2
    GROUPS = FULL // KR
    assert FULL % KR == 0 and GROUPS >= 3

    mesh = plsc.VectorSubcoreMesh(**_MESH)

    @functools.partial(
        pl.kernel, mesh=mesh,
        out_type=jax.ShapeDtypeStruct((NC, N, Dm), jnp.float32),
        scratch_types=(
            [pltpu.VMEM((EK,), jnp.int32) for _ in range(KR)] +
            [pltpu.VMEM((EK,), jnp.int32) for _ in range(KR)] +
            [pltpu.VMEM((EK, Dm), jnp.float32) for _ in range(KR)] +
            [pltpu.VMEM_SHARED((N, Dm), jnp.float32)] +
            [pltpu.SemaphoreType.DMA] * (4 * KR)))
    def scat_kernel(h_hbm, edges_hbm, z_hbm, out_hbm, *rest):
        src_hbm = edges_hbm.at[0]
        dst_hbm = edges_hbm.at[1]
        si = rest[0:KR]
        di = rest[KR:2 * KR]
        rows = rest[2 * KR:3 * KR]
        acc_sh = rest[3 * KR]
        iss = rest[3 * KR + 1:3 * KR + 1 + KR]
        dss = rest[3 * KR + 1 + KR:3 * KR + 1 + 2 * KR]
        gss = rest[3 * KR + 1 + 2 * KR:3 * KR + 1 + 3 * KR]
        sss = rest[3 * KR + 1 + 3 * KR:]
        cid = lax.axis_index("c")
        sid = lax.axis_index("s")
        w = cid * NS + sid

        def e0(c):
            return (w + NT * c) * EK

        def fire_si(c, b):
            pltpu.async_copy(src_hbm.at[pl.ds(e0(c), EK)], si[b], iss[b])

        def wait_si(c, b):
            pltpu.make_async_copy(src_hbm.at[pl.ds(e0(c), EK)], si[b],
                                  iss[b]).wait()

        def fire_di(c, b):
            pltpu.async_copy(dst_hbm.at[pl.ds(e0(c), EK)], di[b], dss[b])

        def wait_di(c, b):
            pltpu.make_async_copy(dst_hbm.at[pl.ds(e0(c), EK)], di[b],
                                  dss[b]).wait()

        def fire_g(b):
            pltpu.async_copy(h_hbm.at[si[b]], rows[b], gss[b])

        def wait_g(b):
            pltpu.make_async_copy(h_hbm.at[si[b]], rows[b], gss[b]).wait()

        def fire_s(b):
            pltpu.async_copy(rows[b], acc_sh.at[di[b]], sss[b], add=True)

        def wait_s(b):
            pltpu.make_async_copy(rows[b], acc_sh.at[di[b]], sss[b]).wait()

        fire_si(0, 0)
        fire_si(1, 1)
        fire_di(0, 0)
        _zero_acc(z_hbm, acc_sh, sid, RPT, TAIL)
        wait_si(0, 0)
        fire_g(0)
        plsc.subcore_barrier()

        def slot(c, b, first=False, fire_next=True, fire_next2=True):
            b1 = 1 - b
            if not first:
                wait_s(b1)
            if fire_next:
                fire_di(c + 1, b1)
                wait_si(c + 1, b1)
                fire_g(b1)
            wait_g(b)
            if fire_next2:
                fire_si(c + 2, b)
            wait_di(c, b)
            fire_s(b)

        slot(0, 0, first=True)
        slot(1, 1)

        @pl.loop(1, GROUPS - 1)
        def _(j):
            slot(KR * j, 0)
            slot(KR * j + 1, 1)

        slot(FULL - 2, 0, fire_next2=False)
        slot(FULL - 1, 1, fire_next=False, fire_next2=False)
        wait_s(1)
        if EXTRA:
            @pl.when(w < EXTRA)
            def _():
                ex0 = (FULL * NT + w) * EK
                pltpu.sync_copy(src_hbm.at[pl.ds(ex0, EK)], si[0])
                pltpu.sync_copy(dst_hbm.at[pl.ds(ex0, EK)], di[0])
                pltpu.sync_copy(h_hbm.at[si[0]], rows[0])
                pltpu.sync_copy(rows[0], acc_sh.at[di[0]], add=True)
        plsc.subcore_barrier()
        _write_acc(acc_sh, out_hbm, cid, sid, RPT, TAIL)

    return scat_kernel(h, edges, z128)


def _dot(a, b):
    return lax.dot_general(a, b, (((1,), (0,)), ((), ())),
                           precision=lax.Precision.DEFAULT,
                           preferred_element_type=jnp.float32)


BM = 5000


def _row_spec(Dm):
    return pl.BlockSpec((BM, Dm), lambda i: (i, 0))


def _pair_spec(Dm):
    return pl.BlockSpec((NC, BM, Dm), lambda i: (0, i, 0))


def _full_spec(a, b):
    return pl.BlockSpec((a, b), lambda i: (0, 0))


def _tc_matmul(x, W):
    N = x.shape[0]

    def body(x_ref, w_ref, o_ref):
        o_ref[...] = _dot(x_ref[...], w_ref[...])

    return pl.pallas_call(
        body,
        grid=(N // BM,),
        in_specs=[_row_spec(x.shape[1]), _full_spec(*W.shape)],
        out_specs=_row_spec(W.shape[1]),
        out_shape=jax.ShapeDtypeStruct((N, W.shape[1]), jnp.float32),
    )(x, W)


def _tc_prep(P1, deg):
    N, Dm = P1.shape

    def body(p_ref, deg_ref, hp_ref, dis_ref):
        d = deg_ref[0][:, 0:1] + deg_ref[1][:, 0:1] + 1.0
        dis = jnp.broadcast_to(lax.rsqrt(d), (BM, Dm))
        dis_ref[...] = dis
        hp_ref[...] = p_ref[...] * dis

    return pl.pallas_call(
        body,
        grid=(N // BM,),
        in_specs=[_row_spec(Dm), _pair_spec(Dm)],
        out_specs=[_row_spec(Dm), _row_spec(Dm)],
        out_shape=[jax.ShapeDtypeStruct((N, Dm), jnp.float32),
                   jax.ShapeDtypeStruct((N, Dm), jnp.float32)],
    )(P1, deg)


def _tc_mid(acc, hp, dis, b1, W2):
    N, Dm = hp.shape

    def body(acc_ref, hp_ref, dis_ref, b_ref, w_ref, o_ref):
        g = jnp.maximum(
            dis_ref[...] * (acc_ref[0] + acc_ref[1] + hp_ref[...])
            + b_ref[...], 0.0)
        o_ref[...] = _dot(g, w_ref[...]) * dis_ref[...]

    return pl.pallas_call(
        body,
        grid=(N // BM,),
        in_specs=[_pair_spec(Dm), _row_spec(Dm), _row_spec(Dm),
                  _full_spec(1, Dm), _full_spec(*W2.shape)],
        out_specs=_row_spec(W2.shape[1]),
        out_shape=jax.ShapeDtypeStruct((N, W2.shape[1]), jnp.float32),
    )(acc, hp, dis, b1, W2)


def _tc_final(acc, hp, dis, b2, Wl, bl):
    N, Dm = hp.shape

    def body(acc_ref, hp_ref, dis_ref, b_ref, w_ref, bl_ref, o_ref):
        g = jnp.maximum(
            dis_ref[...] * (acc_ref[0] + acc_ref[1] + hp_ref[...])
            + b_ref[...], 0.0)
        o_ref[...] = _dot(g, w_ref[...]) + bl_ref[...]

    return pl.pallas_call(
        body,
        grid=(N // BM,),
        in_specs=[_pair_spec(Dm), _row_spec(Dm), _row_spec(Dm),
                  _full_spec(1, Dm), _full_spec(*Wl.shape),
                  _full_spec(1, Wl.shape[1])],
        out_specs=_row_spec(Wl.shape[1]),
        out_shape=jax.ShapeDtypeStruct((N, Wl.shape[1]), jnp.float32),
    )(acc, hp, dis, b2, Wl, bl)


def kernel(x, edge_index, W1, b1, W2, b2, Wl, bl):
    N, D = x.shape
    edges = edge_index if edge_index.dtype == jnp.int32 \
        else edge_index.astype(jnp.int32)
    ones128 = jnp.asarray(np.ones((EK, 128), np.float32))
    z128 = jnp.asarray(np.zeros((N, W1.shape[1]), np.float32))

    deg = _sc_degree(edges, z128, ones128)
    P1 = _tc_matmul(x, W1)
    h1p, disb = _tc_prep(P1, deg)
    acc1 = _sc_scatter(h1p, edges, z128)
    h2p = _tc_mid(acc1, h1p, disb, b1.reshape(1, -1), W2)
    acc2 = _sc_scatter(h2p, edges, z128)
    out = _tc_final(acc2, h2p, disb, b2.reshape(1, -1),
                    Wl, bl.reshape(1, -1))
    return out


# speedup vs baseline: 1.0477x; 1.0192x over previous
import functools

import jax
import jax.numpy as jnp
import numpy as np
from jax import lax
from jax.experimental import pallas as pl
from jax.experimental.pallas import tpu as pltpu
from jax.experimental.pallas import tpu_sc as plsc

NC = 2
NS = 16
EK = 128
R = 4

_MESH = dict(core_axis_name="c", subcore_axis_name="s")


def _zero_acc(z_hbm, acc_sh, sid, RPT, TAIL):
    r0 = sid * RPT
    pltpu.sync_copy(z_hbm.at[pl.ds(r0, RPT)], acc_sh.at[pl.ds(r0, RPT)])
    if TAIL:
        @pl.when(sid == NS - 1)
        def _():
            pltpu.sync_copy(z_hbm.at[pl.ds(NS * RPT, TAIL)],
                            acc_sh.at[pl.ds(NS * RPT, TAIL)])


def _write_acc(acc_sh, out_hbm, cid, sid, RPT, TAIL):
    r0 = sid * RPT
    pltpu.sync_copy(acc_sh.at[pl.ds(r0, RPT)],
                    out_hbm.at[cid, pl.ds(r0, RPT)])
    if TAIL:
        @pl.when(sid == NS - 1)
        def _():
            pltpu.sync_copy(acc_sh.at[pl.ds(NS * RPT, TAIL)],
                            out_hbm.at[cid, pl.ds(NS * RPT, TAIL)])


def _sc_degree(edges, z128, ones128):
    _, E = edges.shape
    N, W = z128.shape
    NT = NC * NS
    NCH = E // EK
    FULL = NCH // NT
    EXTRA = NCH - FULL * NT
    RPT = (N // NS) // 8 * 8
    TAIL = N - NS * RPT
    CH = FULL // R

    mesh = plsc.VectorSubcoreMesh(**_MESH)

    @functools.partial(
        pl.kernel, mesh=mesh,
        out_type=jax.ShapeDtypeStruct((NC, N, W), jnp.float32),
        scratch_types=[
            pltpu.VMEM((FULL, EK), jnp.int32),
            pltpu.VMEM((EK,), jnp.int32),
            pltpu.VMEM((EK, W), jnp.float32),
            pltpu.VMEM_SHARED((N, W), jnp.float32),
            pltpu.SemaphoreType.DMA,
        ] + [pltpu.SemaphoreType.DMA] * R)
    def deg_kernel(edges_hbm, z_hbm, ones_hbm, out_hbm,
                   di2, dix_v, ones_v, acc_sh, m, *ss):
        dst_hbm = edges_hbm.at[1]
        cid = lax.axis_index("c")
        sid = lax.axis_index("s")
        w = cid * NS + sid
        descs = [pltpu.async_copy(dst_hbm.at[pl.ds((w + NT * j) * EK, EK)],
                                  di2.at[j], m) for j in range(FULL)]
        _zero_acc(z_hbm, acc_sh, sid, RPT, TAIL)
        pltpu.sync_copy(ones_hbm.at[pl.ds(0, EK)], ones_v)
        for d in descs:
            d.wait()
        plsc.subcore_barrier()

        for c in range(R):
            pltpu.async_copy(ones_v, acc_sh.at[di2.at[c]], ss[c], add=True)

        @pl.loop(1, CH)
        def _(j):
            for b in range(R):
                c = j * R + b
                pltpu.make_async_copy(ones_v, acc_sh.at[di2.at[c - R]],
                                      ss[b]).wait()
                pltpu.async_copy(ones_v, acc_sh.at[di2.at[c]], ss[b],
                                 add=True)

        for c in range(CH * R, FULL):
            pltpu.make_async_copy(ones_v, acc_sh.at[di2.at[c - R]],
                                  ss[c % R]).wait()
            pltpu.async_copy(ones_v, acc_sh.at[di2.at[c]], ss[c % R],
                             add=True)
        for c in range(FULL - R, FULL):
            pltpu.make_async_copy(ones_v, acc_sh.at[di2.at[c]],
                                  ss[c % R]).wait()
        if EXTRA:
            @pl.when(w < EXTRA)
            def _():
                pltpu.sync_copy(
                    dst_hbm.at[pl.ds((FULL * NT + w) * EK, EK)], dix_v)
                pltpu.sync_copy(ones_v, acc_sh.at[dix_v], add=True)
        plsc.subcore_barrier()
        _write_acc(acc_sh, out_hbm, cid, sid, RPT, TAIL)

    return deg_kernel(edges, z128, ones128)


def _sc_scatter(h, edges, z128):
    N, Dm = h.shape
    _, E = edges.shape
    NT = NC * NS
    NCH = E // EK
    FULL = NCH // NT
    EXTRA = NCH - FULL * NT
    RPT = (N // NS) // 8 * 8
    TAIL = N - NS * RPT
    KR = 2
    GROUPS = FULL // KR
    assert FULL % KR == 0 and GROUPS >= 3

    mesh = plsc.VectorSubcoreMesh(**_MESH)

    @functools.partial(
        pl.kernel, mesh=mesh,
        out_type=jax.ShapeDtypeStruct((NC, N, Dm), jnp.float32),
        scratch_types=(
            [pltpu.VMEM((EK,), jnp.int32) for _ in range(KR)] +
            [pltpu.VMEM((EK,), jnp.int32) for _ in range(KR)] +
            [pltpu.VMEM((EK, Dm), jnp.float32) for _ in range(KR)] +
            [pltpu.VMEM_SHARED((N, Dm), jnp.float32)] +
            [pltpu.SemaphoreType.DMA] * (4 * KR)))
    def scat_kernel(h_hbm, edges_hbm, z_hbm, out_hbm, *rest):
        src_hbm = edges_hbm.at[0]
        dst_hbm = edges_hbm.at[1]
        si = rest[0:KR]
        di = rest[KR:2 * KR]
        rows = rest[2 * KR:3 * KR]
        acc_sh = rest[3 * KR]
        iss = rest[3 * KR + 1:3 * KR + 1 + KR]
        dss = rest[3 * KR + 1 + KR:3 * KR + 1 + 2 * KR]
        gss = rest[3 * KR + 1 + 2 * KR:3 * KR + 1 + 3 * KR]
        sss = rest[3 * KR + 1 + 3 * KR:]
        cid = lax.axis_index("c")
        sid = lax.axis_index("s")
        w = cid * NS + sid

        def e0(c):
            return (w + NT * c) * EK

        def fire_si(c, b):
            pltpu.async_copy(src_hbm.at[pl.ds(e0(c), EK)], si[b], iss[b])

        def wait_si(c, b):
            pltpu.make_async_copy(src_hbm.at[pl.ds(e0(c), EK)], si[b],
                                  iss[b]).wait()

        def fire_di(c, b):
            pltpu.async_copy(dst_hbm.at[pl.ds(e0(c), EK)], di[b], dss[b])

        def wait_di(c, b):
            pltpu.make_async_copy(dst_hbm.at[pl.ds(e0(c), EK)], di[b],
                                  dss[b]).wait()

        def fire_g(b):
            pltpu.async_copy(h_hbm.at[si[b]], rows[b], gss[b])

        def wait_g(b):
            pltpu.make_async_copy(h_hbm.at[si[b]], rows[b], gss[b]).wait()

        def fire_s(b):
            pltpu.async_copy(rows[b], acc_sh.at[di[b]], sss[b], add=True)

        def wait_s(b):
            pltpu.make_async_copy(rows[b], acc_sh.at[di[b]], sss[b]).wait()

        fire_si(0, 0)
        fire_si(1, 1)
        fire_di(0, 0)
        _zero_acc(z_hbm, acc_sh, sid, RPT, TAIL)
        wait_si(0, 0)
        fire_g(0)
        plsc.subcore_barrier()

        def slot(c, b, first=False, fire_next=True, fire_next2=True):
            b1 = 1 - b
            if not first:
                wait_s(b1)
            if fire_next:
                fire_di(c + 1, b1)
                wait_si(c + 1, b1)
                fire_g(b1)
            wait_g(b)
            if fire_next2:
                fire_si(c + 2, b)
            wait_di(c, b)
            fire_s(b)

        slot(0, 0, first=True)
        slot(1, 1)

        @pl.loop(1, GROUPS - 1)
        def _(j):
            slot(KR * j, 0)
            slot(KR * j + 1, 1)

        slot(FULL - 2, 0, fire_next2=False)
        slot(FULL - 1, 1, fire_next=False, fire_next2=False)
        wait_s(1)
        if EXTRA:
            @pl.when(w < EXTRA)
            def _():
                ex0 = (FULL * NT + w) * EK
                pltpu.sync_copy(src_hbm.at[pl.ds(ex0, EK)], si[0])
                pltpu.sync_copy(dst_hbm.at[pl.ds(ex0, EK)], di[0])
                pltpu.sync_copy(h_hbm.at[si[0]], rows[0])
                pltpu.sync_copy(rows[0], acc_sh.at[di[0]], add=True)
        plsc.subcore_barrier()
        _write_acc(acc_sh, out_hbm, cid, sid, RPT, TAIL)

    return scat_kernel(h, edges, z128)


def _dot(a, b):
    return lax.dot_general(a, b, (((1,), (0,)), ((), ())),
                           precision=lax.Precision.DEFAULT,
                           preferred_element_type=jnp.float32)


BM = 5000


def _row_spec(Dm):
    return pl.BlockSpec((BM, Dm), lambda i: (i, 0))


def _pair_spec(Dm):
    return pl.BlockSpec((NC, BM, Dm), lambda i: (0, i, 0))


def _full_spec(a, b):
    return pl.BlockSpec((a, b), lambda i: (0, 0))


def _tc_matmul(x, W):
    N = x.shape[0]

    def body(x_ref, w_ref, o_ref):
        o_ref[...] = _dot(x_ref[...], w_ref[...])

    return pl.pallas_call(
        body,
        grid=(N // BM,),
        in_specs=[_row_spec(x.shape[1]), _full_spec(*W.shape)],
        out_specs=_row_spec(W.shape[1]),
        out_shape=jax.ShapeDtypeStruct((N, W.shape[1]), jnp.float32),
    )(x, W)


def _tc_prep(P1, deg):
    N, Dm = P1.shape

    def body(p_ref, deg_ref, hp_ref, dis_ref):
        d = deg_ref[0][:, 0:1] + deg_ref[1][:, 0:1] + 1.0
        dis = jnp.broadcast_to(lax.rsqrt(d), (BM, Dm))
        dis_ref[...] = dis
        hp_ref[...] = p_ref[...] * dis

    return pl.pallas_call(
        body,
        grid=(N // BM,),
        in_specs=[_row_spec(Dm), _pair_spec(Dm)],
        out_specs=[_row_spec(Dm), _row_spec(Dm)],
        out_shape=[jax.ShapeDtypeStruct((N, Dm), jnp.float32),
                   jax.ShapeDtypeStruct((N, Dm), jnp.float32)],
    )(P1, deg)


def _tc_mid(acc, hp, dis, b1, W2):
    N, Dm = hp.shape

    def body(acc_ref, hp_ref, dis_ref, b_ref, w_ref, o_ref):
        g = jnp.maximum(
            dis_ref[...] * (acc_ref[0] + acc_ref[1] + hp_ref[...])
            + b_ref[...], 0.0)
        o_ref[...] = _dot(g, w_ref[...]) * dis_ref[...]

    return pl.pallas_call(
        body,
        grid=(N // BM,),
        in_specs=[_pair_spec(Dm), _row_spec(Dm), _row_spec(Dm),
                  _full_spec(1, Dm), _full_spec(*W2.shape)],
        out_specs=_row_spec(W2.shape[1]),
        out_shape=jax.ShapeDtypeStruct((N, W2.shape[1]), jnp.float32),
    )(acc, hp, dis, b1, W2)


def _tc_final(acc, hp, dis, b2, Wl, bl):
    N, Dm = hp.shape
    OUT = Wl.shape[1]

    def body(acc_ref, hp_ref, dis_ref, b_ref, w_ref, bl_ref, o_ref):
        g = jnp.maximum(
            dis_ref[...] * (acc_ref[0] + acc_ref[1] + hp_ref[...])
            + b_ref[...], 0.0)
        o_ref[...] = lax.dot_general(
            w_ref[...], g, (((0,), (1,)), ((), ())),
            precision=lax.Precision.DEFAULT,
            preferred_element_type=jnp.float32) + bl_ref[...]

    return pl.pallas_call(
        body,
        in_specs=[pl.BlockSpec((NC, N, Dm), lambda: (0, 0, 0)),
                  pl.BlockSpec((N, Dm), lambda: (0, 0)),
                  pl.BlockSpec((N, Dm), lambda: (0, 0)),
                  pl.BlockSpec((1, Dm), lambda: (0, 0)),
                  pl.BlockSpec(Wl.shape, lambda: (0, 0)),
                  pl.BlockSpec((OUT, 1), lambda: (0, 0))],
        out_specs=pl.BlockSpec((OUT, N), lambda: (0, 0)),
        out_shape=jax.ShapeDtypeStruct((OUT, N), jnp.float32),
    )(acc, hp, dis, b2, Wl, bl)


def kernel(x, edge_index, W1, b1, W2, b2, Wl, bl):
    N, D = x.shape
    edges = edge_index if edge_index.dtype == jnp.int32 \
        else edge_index.astype(jnp.int32)
    ones128 = jnp.asarray(np.ones((EK, 128), np.float32))
    z128 = jnp.asarray(np.zeros((N, W1.shape[1]), np.float32))

    deg = _sc_degree(edges, z128, ones128)
    P1 = _tc_matmul(x, W1)
    h1p, disb = _tc_prep(P1, deg)
    acc1 = _sc_scatter(h1p, edges, z128)
    h2p = _tc_mid(acc1, h1p, disb, b1.reshape(1, -1), W2)
    acc2 = _sc_scatter(h2p, edges, z128)
    outT = _tc_final(acc2, h2p, disb, b2.reshape(1, -1),
                     Wl, bl.reshape(-1, 1))
    return outT.T
